# baseline (speedup 1.0000x reference)
import math

import jax
import jax.numpy as jnp
from jax.experimental import pallas as pl
from jax.experimental.pallas import tpu as pltpu

_DIM = 32
_K = 8192
_LEVELS = 3
_BETA = 0.25
_USAGE_REG = 0.001
_EPS = 1e-05
_TN = 512
_CH = 128
_NCH = _K // _CH
_AUG = 99


def _rvq_body(x_ref, cb_ref, out_ref, codes_ref, commit_ref, usage_ref,
              colsum_acc, commit_acc, wn_ref, waug_ref, qi_ref, *, n_tokens):
    i = pl.program_id(0)
    nt = pl.num_programs(0)

    @pl.when(i == 0)
    def _init():
        colsum_acc[...] = jnp.zeros_like(colsum_acc)
        commit_acc[...] = jnp.zeros_like(commit_acc)
        iota = jax.lax.broadcasted_iota(jnp.int32, (1, _K), 1).astype(jnp.float32)
        hi = jnp.floor(iota * (1.0 / 256.0))
        lo = iota - 256.0 * hi
        for level in range(_LEVELS):
            wt = cb_ref[level].T
            wn_ref[level:level + 1, :] = jnp.sum(wt * wt, axis=0, keepdims=True)
            w1 = wt.astype(jnp.bfloat16)
            rem = wt - w1.astype(jnp.float32)
            w2 = rem.astype(jnp.bfloat16)
            w3 = (rem - w2.astype(jnp.float32)).astype(jnp.bfloat16)
            waug_ref[level, 0:_DIM] = w1
            waug_ref[level, _DIM:2 * _DIM] = w2
            waug_ref[level, 2 * _DIM:3 * _DIM] = w3
            waug_ref[level, 3 * _DIM:3 * _DIM + 1] = hi.astype(jnp.bfloat16)
            waug_ref[level, 3 * _DIM + 1:3 * _DIM + 2] = lo.astype(jnp.bfloat16)
            waug_ref[level, 3 * _DIM + 2:3 * _DIM + 3] = jnp.ones(
                (1, _K), jnp.bfloat16)

    xt = x_ref[...]
    tn = xt.shape[0]
    r = xt
    qsum = jnp.zeros_like(xt)
    commit = jnp.float32(0.0)
    rn = jnp.sum(r * r, axis=1, keepdims=True)
    codes = []
    for level in range(_LEVELS):
        w1 = waug_ref[level, 0:_DIM]
        wn = wn_ref[level:level + 1, :]
        xw2 = jax.lax.dot_general((r * -2.0).astype(jnp.bfloat16),
                                  w1, (((1,), (0,)), ((), ())),
                                  preferred_element_type=jnp.float32)
        d2c = []
        dmin = None
        for c in range(_NCH):
            sl = slice(c * _CH, (c + 1) * _CH)
            dc = (rn + wn[:, sl]) + xw2[:, sl]
            d2c.append(dc)
            m = jnp.min(dc, axis=1, keepdims=True)
            dmin = m if dmin is None else jnp.minimum(dmin, m)
        pc = []
        ohc = []
        z = None
        for c in range(_NCH):
            dc = d2c[c]
            e = jnp.exp(dmin - dc)
            pc.append(e)
            s = jnp.sum(e, axis=1, keepdims=True)
            z = s if z is None else z + s
            ohc.append((dc <= dmin).astype(jnp.bfloat16))
        onehot = jnp.concatenate(ohc, axis=1)
        aug = jax.lax.dot_general(onehot, waug_ref[level],
                                  (((1,), (1,)), ((), ())),
                                  preferred_element_type=jnp.float32)
        q = aug[:, 0:_DIM] + aug[:, _DIM:2 * _DIM] + aug[:, 2 * _DIM:3 * _DIM]
        idxf = aug[:, 3 * _DIM:3 * _DIM + 1] * 256.0 + \
            aug[:, 3 * _DIM + 1:3 * _DIM + 2]
        cnt = aug[:, 3 * _DIM + 2:3 * _DIM + 3]
        qi_ref[:, 0:_DIM] = q
        qi_ref[:, _DIM:_DIM + 1] = idxf

        rz = 1.0 / z
        for c in range(_NCH):
            sl = slice(c * _CH, (c + 1) * _CH)
            colsum_acc[level:level + 1, sl] += jnp.sum(
                pc[c] * rz, axis=0, keepdims=True)

        @pl.when(jnp.max(cnt) > 1.5)
        def _repair_ties():
            d2full = (rn + wn) + xw2
            iota = jax.lax.broadcasted_iota(jnp.int32, (tn, _K), 1)
            idx2 = jnp.min(jnp.where(d2full <= dmin, iota, _K),
                           axis=1, keepdims=True)
            oh2 = (iota == idx2).astype(jnp.bfloat16)
            aug2 = jax.lax.dot_general(oh2, waug_ref[level],
                                       (((1,), (1,)), ((), ())),
                                       preferred_element_type=jnp.float32)
            qi_ref[:, 0:_DIM] = (aug2[:, 0:_DIM] + aug2[:, _DIM:2 * _DIM]
                                 + aug2[:, 2 * _DIM:3 * _DIM])
            qi_ref[:, _DIM:_DIM + 1] = idx2.astype(jnp.float32)

        qv = qi_ref[:, 0:_DIM]
        codes.append(qi_ref[:, _DIM:_DIM + 1].astype(jnp.int32))
        r = r - qv
        rn = jnp.sum(r * r, axis=1, keepdims=True)
        commit = commit + jnp.sum(rn)
        qsum = qsum + qv
    commit_acc[...] += jnp.full((1, 1), commit, jnp.float32)
    out_ref[...] = xt + (qsum - xt)
    codes_ref[...] = jnp.concatenate(codes, axis=1)

    @pl.when(i == nt - 1)
    def _fin():
        commit_ref[...] = commit_acc[...] * (_BETA / (_LEVELS * n_tokens * _DIM))
        ap = jnp.maximum(colsum_acc[...] * (1.0 / n_tokens), _EPS)
        ent = -jnp.sum(ap * jnp.log(ap), keepdims=True)
        usage_ref[...] = (_USAGE_REG / _LEVELS) * (_LEVELS * math.log(_K) - ent)


def _rvq_call(xf, codebooks, interpret=False):
    n = xf.shape[0]
    nt = n // _TN
    body = lambda *refs: _rvq_body(*refs, n_tokens=n)
    out, codes, commit, usage = pl.pallas_call(
        body,
        grid=(nt,),
        in_specs=[
            pl.BlockSpec((_TN, _DIM), lambda i: (i, 0)),
            pl.BlockSpec((_LEVELS, _K, _DIM), lambda i: (0, 0, 0)),
        ],
        out_specs=[
            pl.BlockSpec((_TN, _DIM), lambda i: (i, 0)),
            pl.BlockSpec((_TN, _LEVELS), lambda i: (i, 0)),
            pl.BlockSpec((1, 1), lambda i: (0, 0)),
            pl.BlockSpec((1, 1), lambda i: (0, 0)),
        ],
        out_shape=[
            jax.ShapeDtypeStruct((n, _DIM), jnp.float32),
            jax.ShapeDtypeStruct((n, _LEVELS), jnp.int32),
            jax.ShapeDtypeStruct((1, 1), jnp.float32),
            jax.ShapeDtypeStruct((1, 1), jnp.float32),
        ],
        scratch_shapes=[
            pltpu.VMEM((_LEVELS, _K), jnp.float32),
            pltpu.VMEM((1, 1), jnp.float32),
            pltpu.VMEM((_LEVELS, _K), jnp.float32),
            pltpu.VMEM((_LEVELS, _AUG, _K), jnp.bfloat16),
            pltpu.VMEM((_TN, _DIM + 1), jnp.float32),
        ],
        compiler_params=pltpu.CompilerParams(
            dimension_semantics=("arbitrary",),
        ),
        interpret=interpret,
    )(xf, codebooks)
    return out, codes, commit, usage


@jax.jit
def kernel(x, codebooks):
    b, t, d = x.shape
    xf = x.reshape(-1, d)
    out, codes, commit, usage = _rvq_call(xf, codebooks)
    return (out.reshape(b, t, d),
            codes.reshape(b, t, _LEVELS).astype(jnp.int64),
            commit[0, 0],
            usage[0, 0])


# speedup vs baseline: 1.3183x; 1.3183x over previous
import math

import jax
import jax.numpy as jnp
from jax.experimental import pallas as pl
from jax.experimental.pallas import tpu as pltpu

_DIM = 32
_K = 8192
_LEVELS = 3
_BETA = 0.25
_USAGE_REG = 0.001
_EPS = 1e-05
_TN = 256
_CH = 128
_NCH = _K // _CH
_AUG = 99


def _rvq_body(x_ref, cb_ref, out_ref, codes_ref, commit_ref, usage_ref,
              colsum_acc, commit_acc, wn_ref, waug_ref, qi_ref, *, n_tokens):
    i = pl.program_id(0)
    nt = pl.num_programs(0)

    @pl.when(i == 0)
    def _init():
        colsum_acc[...] = jnp.zeros_like(colsum_acc)
        commit_acc[...] = jnp.zeros_like(commit_acc)
        iota = jax.lax.broadcasted_iota(jnp.int32, (1, _K), 1).astype(jnp.float32)
        hi = jnp.floor(iota * (1.0 / 256.0))
        lo = iota - 256.0 * hi
        for level in range(_LEVELS):
            wt = cb_ref[level].T
            wn_ref[level:level + 1, :] = jnp.sum(wt * wt, axis=0, keepdims=True)
            w1 = wt.astype(jnp.bfloat16)
            rem = wt - w1.astype(jnp.float32)
            w2 = rem.astype(jnp.bfloat16)
            w3 = (rem - w2.astype(jnp.float32)).astype(jnp.bfloat16)
            waug_ref[level, 0:_DIM] = w1
            waug_ref[level, _DIM:2 * _DIM] = w2
            waug_ref[level, 2 * _DIM:3 * _DIM] = w3
            waug_ref[level, 3 * _DIM:3 * _DIM + 1] = hi.astype(jnp.bfloat16)
            waug_ref[level, 3 * _DIM + 1:3 * _DIM + 2] = lo.astype(jnp.bfloat16)
            waug_ref[level, 3 * _DIM + 2:3 * _DIM + 3] = jnp.ones(
                (1, _K), jnp.bfloat16)

    xt = x_ref[...]
    tn = xt.shape[0]
    r = xt
    qsum = jnp.zeros_like(xt)
    commit = jnp.float32(0.0)
    rn = jnp.sum(r * r, axis=1, keepdims=True)
    codes = []
    for level in range(_LEVELS):
        w1 = waug_ref[level, 0:_DIM]
        wn = wn_ref[level:level + 1, :]
        xw2 = jax.lax.dot_general((r * -2.0).astype(jnp.bfloat16),
                                  w1, (((1,), (0,)), ((), ())),
                                  preferred_element_type=jnp.float32)
        d2c = []
        dmin = None
        for c in range(_NCH):
            sl = slice(c * _CH, (c + 1) * _CH)
            dc = (rn + wn[:, sl]) + xw2[:, sl]
            d2c.append(dc)
            m = jnp.min(dc, axis=1, keepdims=True)
            dmin = m if dmin is None else jnp.minimum(dmin, m)
        pc = []
        ohc = []
        z = None
        for c in range(_NCH):
            dc = d2c[c]
            e = jnp.exp(dmin - dc)
            pc.append(e)
            s = jnp.sum(e, axis=1, keepdims=True)
            z = s if z is None else z + s
            ohc.append((dc <= dmin).astype(jnp.bfloat16))
        onehot = jnp.concatenate(ohc, axis=1)
        aug = jax.lax.dot_general(onehot, waug_ref[level],
                                  (((1,), (1,)), ((), ())),
                                  preferred_element_type=jnp.float32)
        q = aug[:, 0:_DIM] + aug[:, _DIM:2 * _DIM] + aug[:, 2 * _DIM:3 * _DIM]
        idxf = aug[:, 3 * _DIM:3 * _DIM + 1] * 256.0 + \
            aug[:, 3 * _DIM + 1:3 * _DIM + 2]
        cnt = aug[:, 3 * _DIM + 2:3 * _DIM + 3]
        qi_ref[:, 0:_DIM] = q
        qi_ref[:, _DIM:_DIM + 1] = idxf

        rz = 1.0 / z
        for c in range(_NCH):
            sl = slice(c * _CH, (c + 1) * _CH)
            colsum_acc[level:level + 1, sl] += jnp.sum(
                pc[c] * rz, axis=0, keepdims=True)

        @pl.when(jnp.max(cnt) > 1.5)
        def _repair_ties():
            d2full = (rn + wn) + xw2
            iota = jax.lax.broadcasted_iota(jnp.int32, (tn, _K), 1)
            idx2 = jnp.min(jnp.where(d2full <= dmin, iota, _K),
                           axis=1, keepdims=True)
            oh2 = (iota == idx2).astype(jnp.bfloat16)
            aug2 = jax.lax.dot_general(oh2, waug_ref[level],
                                       (((1,), (1,)), ((), ())),
                                       preferred_element_type=jnp.float32)
            qi_ref[:, 0:_DIM] = (aug2[:, 0:_DIM] + aug2[:, _DIM:2 * _DIM]
                                 + aug2[:, 2 * _DIM:3 * _DIM])
            qi_ref[:, _DIM:_DIM + 1] = idx2.astype(jnp.float32)

        qv = qi_ref[:, 0:_DIM]
        codes.append(qi_ref[:, _DIM:_DIM + 1].astype(jnp.int32))
        r = r - qv
        rn = jnp.sum(r * r, axis=1, keepdims=True)
        commit = commit + jnp.sum(rn)
        qsum = qsum + qv
    commit_acc[...] += jnp.full((1, 1), commit, jnp.float32)
    out_ref[...] = xt + (qsum - xt)
    codes_ref[...] = jnp.concatenate(codes, axis=1)

    @pl.when(i == nt - 1)
    def _fin():
        commit_ref[...] = commit_acc[...] * (_BETA / (_LEVELS * n_tokens * _DIM))
        ap = jnp.maximum(colsum_acc[...] * (1.0 / n_tokens), _EPS)
        ent = -jnp.sum(ap * jnp.log(ap), keepdims=True)
        usage_ref[...] = (_USAGE_REG / _LEVELS) * (_LEVELS * math.log(_K) - ent)


def _rvq_call(xf, codebooks, interpret=False):
    n = xf.shape[0]
    nt = n // _TN
    body = lambda *refs: _rvq_body(*refs, n_tokens=n)
    out, codes, commit, usage = pl.pallas_call(
        body,
        grid=(nt,),
        in_specs=[
            pl.BlockSpec((_TN, _DIM), lambda i: (i, 0)),
            pl.BlockSpec((_LEVELS, _K, _DIM), lambda i: (0, 0, 0)),
        ],
        out_specs=[
            pl.BlockSpec((_TN, _DIM), lambda i: (i, 0)),
            pl.BlockSpec((_TN, _LEVELS), lambda i: (i, 0)),
            pl.BlockSpec((1, 1), lambda i: (0, 0)),
            pl.BlockSpec((1, 1), lambda i: (0, 0)),
        ],
        out_shape=[
            jax.ShapeDtypeStruct((n, _DIM), jnp.float32),
            jax.ShapeDtypeStruct((n, _LEVELS), jnp.int32),
            jax.ShapeDtypeStruct((1, 1), jnp.float32),
            jax.ShapeDtypeStruct((1, 1), jnp.float32),
        ],
        scratch_shapes=[
            pltpu.VMEM((_LEVELS, _K), jnp.float32),
            pltpu.VMEM((1, 1), jnp.float32),
            pltpu.VMEM((_LEVELS, _K), jnp.float32),
            pltpu.VMEM((_LEVELS, _AUG, _K), jnp.bfloat16),
            pltpu.VMEM((_TN, _DIM + 1), jnp.float32),
        ],
        compiler_params=pltpu.CompilerParams(
            dimension_semantics=("arbitrary",),
        ),
        interpret=interpret,
    )(xf, codebooks)
    return out, codes, commit, usage


@jax.jit
def kernel(x, codebooks):
    b, t, d = x.shape
    xf = x.reshape(-1, d)
    out, codes, commit, usage = _rvq_call(xf, codebooks)
    return (out.reshape(b, t, d),
            codes.reshape(b, t, _LEVELS).astype(jnp.int64),
            commit[0, 0],
            usage[0, 0])
